# baseline (speedup 1.0000x reference)
import jax
import jax.numpy as jnp
from jax import lax
from jax.experimental import pallas as pl
from jax.experimental.pallas import tpu as pltpu

EDGE_TILE = 32768


def _round_up(x, m):
    return ((x + m - 1) // m) * m


def _dg(lhs, rhs, dims):
    return lax.dot_general(lhs, rhs, dimension_numbers=(dims, ((), ())),
                           preferred_element_type=jnp.float32)


def kernel(emb_w, tag_w, per_w, grp_w, lin_w, lin_b, lin_e_w, lin_e_b,
           period_table, group_table, z, tag, rel_pos, edge_attr):
    n = z.shape[0]
    e = rel_pos.shape[0]
    n_elements = emb_w.shape[0]
    n_tags = tag_w.shape[0]
    atom_dim = emb_w.shape[1]
    tag_dim = tag_w.shape[1]
    pg_dim = per_w.shape[1]
    n_periods = per_w.shape[0]
    n_groups = grp_w.shape[0]
    hidden = lin_w.shape[1]
    rp_dim = rel_pos.shape[1]
    ea_dim = edge_attr.shape[1]
    vocab = n_elements + n_tags

    def fused_kernel(z_ref, t_ref, pt_ref, gt_ref, emb_ref, tagw_ref, per_ref,
                     grp_ref, lw_ref, lb_ref, lew_ref, leb_ref,
                     rp_ref, ea_ref, ht_ref, et_ref):
        f32 = jnp.float32
        lw = lw_ref[...]
        o = atom_dim + tag_dim
        emb_efft = _dg(lw[:atom_dim], emb_ref[...], ((0,), (1,)))
        tag_efft = _dg(lw[atom_dim:o], tagw_ref[...], ((0,), (1,)))
        per_efft = _dg(lw[o:o + pg_dim], per_ref[...], ((0,), (1,)))
        grp_efft = _dg(lw[o + pg_dim:], grp_ref[...], ((0,), (1,)))
        pmask = (lax.broadcasted_iota(jnp.int32, (n_periods, n_elements), 0)
                 == pt_ref[...][None, :]).astype(f32)
        gmask = (lax.broadcasted_iota(jnp.int32, (n_groups, n_elements), 0)
                 == gt_ref[...][None, :]).astype(f32)
        lb_col = jnp.swapaxes(lb_ref[...], 0, 1)
        at = (emb_efft
              + jnp.dot(per_efft, pmask, preferred_element_type=f32)
              + jnp.dot(grp_efft, gmask, preferred_element_type=f32)
              + lb_col)
        ct = jnp.concatenate([at, tag_efft], axis=1)

        cols = z_ref.shape[0]
        vrow = lax.broadcasted_iota(jnp.int32, (vocab, cols), 0)
        zrow = jnp.broadcast_to(z_ref[...][None, :], (vocab, cols))
        trow = jnp.broadcast_to(t_ref[...][None, :] + n_elements, (vocab, cols))
        mh = ((vrow == zrow) | (vrow == trow)).astype(f32)
        ht_ref[...] = jnp.dot(ct, mh, preferred_element_type=f32)

        lew = lew_ref[...]
        leb_col = jnp.swapaxes(leb_ref[...], 0, 1)
        et_ref[...] = (_dg(lew[:rp_dim], rp_ref[...], ((0,), (0,)))
                       + _dg(lew[rp_dim:], ea_ref[...], ((0,), (0,)))
                       + leb_col)

    rp_t = rel_pos.astype(jnp.float32).T
    ea_t = edge_attr.astype(jnp.float32).T
    zc = z.astype(jnp.int32)
    tc = tag.astype(jnp.int32)

    e_pad = _round_up(max(e, 1), EDGE_TILE)
    g = e_pad // EDGE_TILE
    tn = _round_up(-(-max(n, 1) // g), 128)
    n_pad = g * tn
    if n_pad != n:
        zc = jnp.pad(zc, (0, n_pad - n))
        tc = jnp.pad(tc, (0, n_pad - n))
    if e_pad != e:
        rp_t = jnp.pad(rp_t, ((0, 0), (0, e_pad - e)))
        ea_t = jnp.pad(ea_t, ((0, 0), (0, e_pad - e)))

    full = lambda i: (0, 0)
    ht, et = pl.pallas_call(
        fused_kernel,
        out_shape=(jax.ShapeDtypeStruct((hidden, n_pad), jnp.float32),
                   jax.ShapeDtypeStruct((hidden, e_pad), jnp.float32)),
        grid=(g,),
        in_specs=[
            pl.BlockSpec((tn,), lambda i: (i,)),
            pl.BlockSpec((tn,), lambda i: (i,)),
            pl.BlockSpec((n_elements,), lambda i: (0,)),
            pl.BlockSpec((n_elements,), lambda i: (0,)),
            pl.BlockSpec((n_elements, atom_dim), full),
            pl.BlockSpec((n_tags, tag_dim), full),
            pl.BlockSpec((n_periods, pg_dim), full),
            pl.BlockSpec((n_groups, pg_dim), full),
            pl.BlockSpec((lin_w.shape[0], hidden), full),
            pl.BlockSpec((1, hidden), full),
            pl.BlockSpec((rp_dim + ea_dim, hidden), full),
            pl.BlockSpec((1, hidden), full),
            pl.BlockSpec((rp_dim, EDGE_TILE), lambda i: (0, i)),
            pl.BlockSpec((ea_dim, EDGE_TILE), lambda i: (0, i)),
        ],
        out_specs=(pl.BlockSpec((hidden, tn), lambda i: (0, i)),
                   pl.BlockSpec((hidden, EDGE_TILE), lambda i: (0, i))),
        compiler_params=pltpu.CompilerParams(
            dimension_semantics=("parallel",)),
    )(zc, tc, period_table.astype(jnp.int32), group_table.astype(jnp.int32),
      emb_w.astype(jnp.float32), tag_w.astype(jnp.float32),
      per_w.astype(jnp.float32), grp_w.astype(jnp.float32),
      lin_w.astype(jnp.float32), lin_b.astype(jnp.float32),
      lin_e_w.astype(jnp.float32), lin_e_b.astype(jnp.float32),
      rp_t, ea_t)

    h = ht.T if n_pad == n else ht[:, :n].T
    e_out = et.T if e_pad == e else et[:, :e].T
    return h, e_out


# speedup vs baseline: 1.0843x; 1.0843x over previous
import jax
import jax.numpy as jnp
from jax import lax
from jax.experimental import pallas as pl
from jax.experimental.pallas import tpu as pltpu

EDGE_TILE = 65536


def _round_up(x, m):
    return ((x + m - 1) // m) * m


def _dg(lhs, rhs, dims):
    return lax.dot_general(lhs, rhs, dimension_numbers=(dims, ((), ())),
                           preferred_element_type=jnp.float32)


def kernel(emb_w, tag_w, per_w, grp_w, lin_w, lin_b, lin_e_w, lin_e_b,
           period_table, group_table, z, tag, rel_pos, edge_attr):
    n = z.shape[0]
    e = rel_pos.shape[0]
    n_elements = emb_w.shape[0]
    n_tags = tag_w.shape[0]
    atom_dim = emb_w.shape[1]
    tag_dim = tag_w.shape[1]
    pg_dim = per_w.shape[1]
    n_periods = per_w.shape[0]
    n_groups = grp_w.shape[0]
    hidden = lin_w.shape[1]
    rp_dim = rel_pos.shape[1]
    ea_dim = edge_attr.shape[1]
    vocab = n_elements + n_tags

    def fused_kernel(z_ref, t_ref, pt_ref, gt_ref, emb_ref, tagw_ref, per_ref,
                     grp_ref, lw_ref, lb_ref, lew_ref, leb_ref,
                     rp_ref, ea_ref, ht_ref, et_ref):
        f32 = jnp.float32
        lw = lw_ref[...]
        o = atom_dim + tag_dim
        emb_efft = _dg(lw[:atom_dim], emb_ref[...], ((0,), (1,)))
        tag_efft = _dg(lw[atom_dim:o], tagw_ref[...], ((0,), (1,)))
        per_efft = _dg(lw[o:o + pg_dim], per_ref[...], ((0,), (1,)))
        grp_efft = _dg(lw[o + pg_dim:], grp_ref[...], ((0,), (1,)))
        pmask = (lax.broadcasted_iota(jnp.int32, (n_periods, n_elements), 0)
                 == pt_ref[...][None, :]).astype(f32)
        gmask = (lax.broadcasted_iota(jnp.int32, (n_groups, n_elements), 0)
                 == gt_ref[...][None, :]).astype(f32)
        lb_col = jnp.swapaxes(lb_ref[...], 0, 1)
        at = (emb_efft
              + jnp.dot(per_efft, pmask, preferred_element_type=f32)
              + jnp.dot(grp_efft, gmask, preferred_element_type=f32)
              + lb_col)
        ct = jnp.concatenate([at, tag_efft], axis=1)

        cols = z_ref.shape[0]
        vrow = lax.broadcasted_iota(jnp.int32, (vocab, cols), 0)
        zrow = jnp.broadcast_to(z_ref[...][None, :], (vocab, cols))
        trow = jnp.broadcast_to(t_ref[...][None, :] + n_elements, (vocab, cols))
        mh = ((vrow == zrow) | (vrow == trow)).astype(f32)
        ht_ref[...] = jnp.dot(ct, mh, preferred_element_type=f32)

        lew = lew_ref[...]
        leb_col = jnp.swapaxes(leb_ref[...], 0, 1)
        et_ref[...] = (_dg(lew[:rp_dim], rp_ref[...], ((0,), (0,)))
                       + _dg(lew[rp_dim:], ea_ref[...], ((0,), (0,)))
                       + leb_col)

    rp_t = rel_pos.astype(jnp.float32).T
    ea_t = edge_attr.astype(jnp.float32).T
    zc = z.astype(jnp.int32)
    tc = tag.astype(jnp.int32)

    e_pad = _round_up(max(e, 1), EDGE_TILE)
    g = e_pad // EDGE_TILE
    tn = _round_up(-(-max(n, 1) // g), 128)
    n_pad = g * tn
    if n_pad != n:
        zc = jnp.pad(zc, (0, n_pad - n))
        tc = jnp.pad(tc, (0, n_pad - n))
    if e_pad != e:
        rp_t = jnp.pad(rp_t, ((0, 0), (0, e_pad - e)))
        ea_t = jnp.pad(ea_t, ((0, 0), (0, e_pad - e)))

    full = lambda i: (0, 0)
    ht, et = pl.pallas_call(
        fused_kernel,
        out_shape=(jax.ShapeDtypeStruct((hidden, n_pad), jnp.float32),
                   jax.ShapeDtypeStruct((hidden, e_pad), jnp.float32)),
        grid=(g,),
        in_specs=[
            pl.BlockSpec((tn,), lambda i: (i,)),
            pl.BlockSpec((tn,), lambda i: (i,)),
            pl.BlockSpec((n_elements,), lambda i: (0,)),
            pl.BlockSpec((n_elements,), lambda i: (0,)),
            pl.BlockSpec((n_elements, atom_dim), full),
            pl.BlockSpec((n_tags, tag_dim), full),
            pl.BlockSpec((n_periods, pg_dim), full),
            pl.BlockSpec((n_groups, pg_dim), full),
            pl.BlockSpec((lin_w.shape[0], hidden), full),
            pl.BlockSpec((1, hidden), full),
            pl.BlockSpec((rp_dim + ea_dim, hidden), full),
            pl.BlockSpec((1, hidden), full),
            pl.BlockSpec((rp_dim, EDGE_TILE), lambda i: (0, i)),
            pl.BlockSpec((ea_dim, EDGE_TILE), lambda i: (0, i)),
        ],
        out_specs=(pl.BlockSpec((hidden, tn), lambda i: (0, i)),
                   pl.BlockSpec((hidden, EDGE_TILE), lambda i: (0, i))),
        compiler_params=pltpu.CompilerParams(
            dimension_semantics=("parallel",)),
    )(zc, tc, period_table.astype(jnp.int32), group_table.astype(jnp.int32),
      emb_w.astype(jnp.float32), tag_w.astype(jnp.float32),
      per_w.astype(jnp.float32), grp_w.astype(jnp.float32),
      lin_w.astype(jnp.float32), lin_b.astype(jnp.float32),
      lin_e_w.astype(jnp.float32), lin_e_b.astype(jnp.float32),
      rp_t, ea_t)

    h = ht.T if n_pad == n else ht[:, :n].T
    e_out = et.T if e_pad == e else et[:, :e].T
    return h, e_out
